# baseline (speedup 1.0000x reference)
import numpy as np

import jax
import jax.numpy as jnp
from jax import lax
from jax.experimental import pallas as pl
from jax.experimental.pallas import tpu as pltpu
from jax.experimental.pallas import tpu_sc as plsc

N = 10000
K = 32
D = 128
OUT = 128
TW = 160

NW = 32
NPAD = 10240
TPW = NPAD // NW
CH = 8
CHK = CH * K
NCH = TPW // CH
DG = D // 16
NB = 3

_PERM = np.empty((D,), np.int32)
for _u in range(D // 32):
    for _t in range(16):
        _PERM[32 * _u + _t] = 32 * _u + 2 * _t
        _PERM[32 * _u + 16 + _t] = 32 * _u + 2 * _t + 1


def _unpack2(x32):
    return plsc.unpack(x32, format=plsc.PackFormat.INTERLEAVED)


def _norm_body(x_ref, o_ref):
    x = x_ref[...]
    o_ref[...] = lax.rsqrt(jnp.sum(x * x, axis=1, keepdims=True))


def _inv_norms(feat):
    rows = 1000
    return pl.pallas_call(
        _norm_body,
        grid=(N // rows,),
        in_specs=[pl.BlockSpec((rows, D), lambda i: (i, 0))],
        out_specs=pl.BlockSpec((rows, 1), lambda i: (i, 0)),
        out_shape=jax.ShapeDtypeStruct((N, 1), jnp.float32),
    )(feat)


def _sc_body(tab_hbm, idx_hbm, agg_hbm, ws_hbm,
             idx_v, ctr_v, rows0, rows1, rows2, out0, out1, out2, ws_v,
             sem0, sem1, sem2, osem0, osem1, osem2):
    wid = lax.axis_index("s") * 2 + lax.axis_index("c")
    base = wid * TPW

    pltpu.sync_copy(idx_hbm.at[pl.ds(base * K, TPW * K)], idx_v)
    pltpu.sync_copy(tab_hbm.at[pl.ds(base, TPW)], ctr_v)

    rows = (rows0, rows1, rows2)
    sems = (sem0, sem1, sem2)
    outs = (out0, out1, out2)
    osems = (osem0, osem1, osem2)

    def start_gather(g, b):
        pltpu.async_copy(tab_hbm.at[idx_v.at[pl.ds(g * CHK, CHK)]],
                         rows[b], sems[b])

    for b in range(NB):
        start_gather(b, b)

    def do_chunk(g, b, first, issue_next):
        pltpu.make_async_copy(tab_hbm.at[pl.ds(0, CHK)],
                              rows[b], sems[b]).wait()
        rv = rows[b]

        if first is None:
            pltpu.make_async_copy(
                outs[b], agg_hbm.at[pl.ds(base, CH)], osems[b]).wait()
        else:
            @pl.when(jnp.logical_not(first))
            def _():
                pltpu.make_async_copy(
                    outs[b], agg_hbm.at[pl.ds(base, CH)], osems[b]).wait()

        for i in range(CH):
            ln = g * CH + i
            ec, _ = _unpack2(ctr_v[ln, pl.ds(D, 32)])
            invnc = plsc.cumsum(ec)
            c = []
            for u in range(DG // 2):
                c0, c1 = _unpack2(ctr_v[ln, pl.ds(32 * u, 32)])
                c.append(c0 * invnc)
                c.append(c1 * invnc)
            zero16 = jnp.zeros((16,), jnp.float32)
            init = (tuple(zero16 for _ in range(DG)), zero16)

            @plsc.parallel_loop(i * K, (i + 1) * K, 1, unroll=4, carry=init)
            def kstep(row, car):
                acc, wsvec = car
                f = []
                for u in range(DG // 2):
                    f0, f1 = _unpack2(rv[row, pl.ds(32 * u, 32)])
                    f.append(f0)
                    f.append(f1)
                dot = ((f[0] * c[0] + f[1] * c[1])
                       + (f[2] * c[2] + f[3] * c[3])) + (
                      (f[4] * c[4] + f[5] * c[5])
                       + (f[6] * c[6] + f[7] * c[7]))
                s_cum = plsc.cumsum(dot)
                e, _ = _unpack2(rv[row, pl.ds(D, 32)])
                w = plsc.cumsum(jnp.flip(s_cum, 0) * e)
                return (tuple(acc[j] + w * f[j] for j in range(DG)),
                        wsvec + w)

            acc, wsvec = kstep
            for j in range(DG):
                outs[b][i, pl.ds(16 * j, 16)] = acc[j]
            ws_v[ln, :] = wsvec

        if issue_next:
            @pl.when(g + NB < NCH)
            def _():
                start_gather(g + NB, b)
        pltpu.async_copy(outs[b], agg_hbm.at[pl.ds(base + g * CH, CH)],
                         osems[b])

    NTRIP = NCH // NB

    def trip(g3, carry):
        for b in range(NB):
            do_chunk(g3 * NB + b, b, first=g3 == 0, issue_next=True)
        return carry

    lax.fori_loop(0, NTRIP, trip, 0)
    for r in range(NCH - NTRIP * NB):
        do_chunk(NTRIP * NB + r, r, first=None, issue_next=False)
    for b in range(NB):
        pltpu.make_async_copy(outs[b], agg_hbm.at[pl.ds(base, CH)],
                              osems[b]).wait()
    pltpu.sync_copy(ws_v, ws_hbm.at[pl.ds(base, TPW)])


def _sc_aggregate(table, idx_flat):
    mesh = plsc.VectorSubcoreMesh(core_axis_name="c", subcore_axis_name="s")
    fn = pl.kernel(
        _sc_body, mesh=mesh,
        out_type=[
            jax.ShapeDtypeStruct((NPAD, D), jnp.float32),
            jax.ShapeDtypeStruct((NPAD, 16), jnp.float32),
        ],
        scratch_types=[
            pltpu.VMEM((TPW * K,), jnp.int32),
            pltpu.VMEM((TPW, TW), jnp.bfloat16),
            pltpu.VMEM((CHK, TW), jnp.bfloat16),
            pltpu.VMEM((CHK, TW), jnp.bfloat16),
            pltpu.VMEM((CHK, TW), jnp.bfloat16),
            pltpu.VMEM((CH, D), jnp.float32),
            pltpu.VMEM((CH, D), jnp.float32),
            pltpu.VMEM((CH, D), jnp.float32),
            pltpu.VMEM((TPW, 16), jnp.float32),
            pltpu.SemaphoreType.DMA,
            pltpu.SemaphoreType.DMA,
            pltpu.SemaphoreType.DMA,
            pltpu.SemaphoreType.DMA,
            pltpu.SemaphoreType.DMA,
            pltpu.SemaphoreType.DMA,
        ],
        compiler_params=pltpu.CompilerParams(
            needs_layout_passes=False, use_tc_tiling_on_sc=False),
    )
    return fn(table, idx_flat)


def _final_body(agg_ref, ws_ref, x_ref, wt_ref, b_ref, o_ref):
    w1 = wt_ref[0:D, :]
    w2 = wt_ref[D:2 * D, :]
    y1 = jnp.dot(agg_ref[...], w1, preferred_element_type=jnp.float32)
    y2 = jnp.dot(x_ref[...], w2, preferred_element_type=jnp.float32) + b_ref[...]
    o_ref[...] = (y1 + ws_ref[...] * y2) * (1.0 / K)


def _final(agg, ws, feat, wt, b2):
    rows = 1000
    return pl.pallas_call(
        _final_body,
        grid=(N // rows,),
        in_specs=[
            pl.BlockSpec((rows, D), lambda i: (i, 0)),
            pl.BlockSpec((rows, 1), lambda i: (i, 0)),
            pl.BlockSpec((rows, D), lambda i: (i, 0)),
            pl.BlockSpec((2 * D, OUT), lambda i: (0, 0)),
            pl.BlockSpec((1, OUT), lambda i: (0, 0)),
        ],
        out_specs=pl.BlockSpec((rows, OUT), lambda i: (i, 0)),
        out_shape=jax.ShapeDtypeStruct((N, OUT), jnp.float32),
    )(agg, ws, feat, wt, b2)


def kernel(feat_prop, neigh_idx, W, b):
    invn = _inv_norms(feat_prop)

    table = jnp.zeros((NPAD, TW), jnp.bfloat16)
    table = table.at[:N, :D].set(feat_prop.astype(jnp.bfloat16))
    table = table.at[:N, D].set(invn[:, 0].astype(jnp.bfloat16))
    idx_flat = jnp.zeros((NPAD, K), jnp.int32).at[:N].set(neigh_idx).reshape(-1)

    agg, ws = _sc_aggregate(table, idx_flat)

    wt = W.T
    wt = jnp.concatenate([wt[:D][_PERM], wt[D:]], axis=0)
    return _final(agg[:N], ws[:N, :1], feat_prop, wt, b.reshape(1, OUT))


# speedup vs baseline: 1.1256x; 1.1256x over previous
import numpy as np

import jax
import jax.numpy as jnp
from jax import lax
from jax.experimental import pallas as pl
from jax.experimental.pallas import tpu as pltpu
from jax.experimental.pallas import tpu_sc as plsc

N = 10000
K = 32
D = 128
OUT = 128
TW = 160

NW = 32
NPAD = 10240
CH = 8
CHK = CH * K
DG = D // 16
NB = 2
NA0 = 384
NA1 = 256
STR = NA0 + NA1
NCH0 = NA0 // CH
NCH1 = NA1 // CH
NROWS = 16 * STR + NA0

_PERM = np.empty((D,), np.int32)
for _u in range(D // 32):
    for _t in range(16):
        _PERM[32 * _u + _t] = 32 * _u + 2 * _t
        _PERM[32 * _u + 16 + _t] = 32 * _u + 2 * _t + 1


def _unpack2(x32):
    return plsc.unpack(x32, format=plsc.PackFormat.INTERLEAVED)


def _norm_body(x_ref, o_ref):
    x = x_ref[...]
    o_ref[...] = lax.rsqrt(jnp.sum(x * x, axis=1, keepdims=True))


def _inv_norms(feat):
    rows = 1000
    return pl.pallas_call(
        _norm_body,
        grid=(N // rows,),
        in_specs=[pl.BlockSpec((rows, D), lambda i: (i, 0))],
        out_specs=pl.BlockSpec((rows, 1), lambda i: (i, 0)),
        out_shape=jax.ShapeDtypeStruct((N, 1), jnp.float32),
    )(feat)


def _sc_body(tab_hbm, idx_hbm, agg_hbm, ws_hbm,
             idx_v, ctr_v, rows0, rows1, out0, out1, ws_v,
             sem0, sem1, osem0, osem1):
    cix = lax.axis_index("c")
    base = lax.axis_index("s") * STR + cix * NA0
    nch = jnp.where(cix == 0, NCH0, NCH1)

    pltpu.sync_copy(idx_hbm.at[pl.ds(base * K, NA0 * K)], idx_v)
    pltpu.sync_copy(tab_hbm.at[pl.ds(base, NA0)], ctr_v)

    rows = (rows0, rows1)
    sems = (sem0, sem1)
    outs = (out0, out1)
    osems = (osem0, osem1)

    def start_gather(g, b):
        pltpu.async_copy(tab_hbm.at[idx_v.at[pl.ds(g * CHK, CHK)]],
                         rows[b], sems[b])

    for b in range(NB):
        start_gather(b, b)

    def do_chunk(g, b, first, issue_next):
        pltpu.make_async_copy(tab_hbm.at[pl.ds(0, CHK)],
                              rows[b], sems[b]).wait()
        rv = rows[b]

        if first is None:
            pltpu.make_async_copy(
                outs[b], agg_hbm.at[pl.ds(base, CH)], osems[b]).wait()
        else:
            @pl.when(jnp.logical_not(first))
            def _():
                pltpu.make_async_copy(
                    outs[b], agg_hbm.at[pl.ds(base, CH)], osems[b]).wait()

        for i in range(CH):
            ln = g * CH + i
            ec, _ = _unpack2(ctr_v[ln, pl.ds(D, 32)])
            invnc = plsc.cumsum(ec)
            c = []
            for u in range(DG // 2):
                c0, c1 = _unpack2(ctr_v[ln, pl.ds(32 * u, 32)])
                c.append(c0 * invnc)
                c.append(c1 * invnc)
            zero16 = jnp.zeros((16,), jnp.float32)
            init = (tuple(zero16 for _ in range(DG)), zero16)

            @plsc.parallel_loop(i * K, (i + 1) * K, 1, unroll=4, carry=init)
            def kstep(row, car):
                acc, wsvec = car
                f = []
                for u in range(DG // 2):
                    f0, f1 = _unpack2(rv[row, pl.ds(32 * u, 32)])
                    f.append(f0)
                    f.append(f1)
                dot = ((f[0] * c[0] + f[1] * c[1])
                       + (f[2] * c[2] + f[3] * c[3])) + (
                      (f[4] * c[4] + f[5] * c[5])
                       + (f[6] * c[6] + f[7] * c[7]))
                s_cum = plsc.cumsum(dot)
                e, _ = _unpack2(rv[row, pl.ds(D, 32)])
                w = plsc.cumsum(jnp.flip(s_cum, 0) * e)
                return (tuple(acc[j] + w * f[j] for j in range(DG)),
                        wsvec + w)

            acc, wsvec = kstep
            for j in range(DG):
                outs[b][i, pl.ds(16 * j, 16)] = acc[j]
            ws_v[ln, :] = wsvec

        if issue_next:
            @pl.when(g + NB < nch)
            def _():
                start_gather(g + NB, b)
        pltpu.async_copy(outs[b], agg_hbm.at[pl.ds(base + g * CH, CH)],
                         osems[b])

    def trip(g3, carry):
        for b in range(NB):
            do_chunk(g3 * NB + b, b, first=g3 == 0, issue_next=True)
        return carry

    lax.fori_loop(0, nch // NB, trip, 0)
    for b in range(NB):
        pltpu.make_async_copy(outs[b], agg_hbm.at[pl.ds(base, CH)],
                              osems[b]).wait()

    @pl.when(cix == 0)
    def _():
        pltpu.sync_copy(ws_v, ws_hbm.at[pl.ds(base, NA0)])

    @pl.when(cix == 1)
    def _():
        pltpu.sync_copy(ws_v.at[pl.ds(0, NA1)], ws_hbm.at[pl.ds(base, NA1)])


def _sc_aggregate(table, idx_flat):
    mesh = plsc.VectorSubcoreMesh(core_axis_name="c", subcore_axis_name="s")
    fn = pl.kernel(
        _sc_body, mesh=mesh,
        out_type=[
            jax.ShapeDtypeStruct((NPAD, D), jnp.float32),
            jax.ShapeDtypeStruct((NPAD, 16), jnp.float32),
        ],
        scratch_types=[
            pltpu.VMEM((NA0 * K,), jnp.int32),
            pltpu.VMEM((NA0, TW), jnp.bfloat16),
            pltpu.VMEM((CHK, TW), jnp.bfloat16),
            pltpu.VMEM((CHK, TW), jnp.bfloat16),
            pltpu.VMEM((CH, D), jnp.float32),
            pltpu.VMEM((CH, D), jnp.float32),
            pltpu.VMEM((NA0, 16), jnp.float32),
            pltpu.SemaphoreType.DMA,
            pltpu.SemaphoreType.DMA,
            pltpu.SemaphoreType.DMA,
            pltpu.SemaphoreType.DMA,
        ],
        compiler_params=pltpu.CompilerParams(
            needs_layout_passes=False, use_tc_tiling_on_sc=False),
    )
    return fn(table, idx_flat)


def _final_body(agg_ref, ws_ref, x_ref, wt_ref, b_ref, o_ref):
    w1 = wt_ref[0:D, :]
    w2 = wt_ref[D:2 * D, :]
    y1 = jnp.dot(agg_ref[...], w1, preferred_element_type=jnp.float32)
    y2 = jnp.dot(x_ref[...], w2, preferred_element_type=jnp.float32) + b_ref[...]
    o_ref[...] = (y1 + ws_ref[...] * y2) * (1.0 / K)


def _final(agg, ws, feat, wt, b2):
    rows = 1000
    return pl.pallas_call(
        _final_body,
        grid=(N // rows,),
        in_specs=[
            pl.BlockSpec((rows, D), lambda i: (i, 0)),
            pl.BlockSpec((rows, 1), lambda i: (i, 0)),
            pl.BlockSpec((rows, D), lambda i: (i, 0)),
            pl.BlockSpec((2 * D, OUT), lambda i: (0, 0)),
            pl.BlockSpec((1, OUT), lambda i: (0, 0)),
        ],
        out_specs=pl.BlockSpec((rows, OUT), lambda i: (i, 0)),
        out_shape=jax.ShapeDtypeStruct((N, OUT), jnp.float32),
    )(agg, ws, feat, wt, b2)


def kernel(feat_prop, neigh_idx, W, b):
    invn = _inv_norms(feat_prop)

    table = jnp.zeros((NROWS, TW), jnp.bfloat16)
    table = table.at[:N, :D].set(feat_prop.astype(jnp.bfloat16))
    table = table.at[:N, D].set(invn[:, 0].astype(jnp.bfloat16))
    idx_flat = jnp.zeros((NROWS, K), jnp.int32).at[:N].set(neigh_idx).reshape(-1)

    agg, ws = _sc_aggregate(table, idx_flat)

    wt = W.T
    wt = jnp.concatenate([wt[:D][_PERM], wt[D:]], axis=0)
    return _final(agg[:N], ws[:N, :1], feat_prop, wt, b.reshape(1, OUT))
